# baseline (speedup 1.0000x reference)
import jax
import jax.numpy as jnp
from jax.experimental import pallas as pl
from jax.experimental.pallas import tpu as pltpu

_NUM_CODES = 8192
_NO_CODE_ID = 8192
_DIST_THRESHOLD = 128.0
_CODE_DIM = 64
_MT = 1024
_CH = 128
_RB = 32


def _vq_kernel(z_ref, codes_ref, codes_t_ref, ids_ref, mind_ref):
    ct = codes_t_ref[...]
    c2 = jnp.sum(ct * ct, axis=0, keepdims=True)

    zf = z_ref[...]
    zn = zf * jnp.float32(-2.0)
    q2 = jax.lax.dot_general(
        zn, codes_ref[...], (((1,), (1,)), ((), ())),
        preferred_element_type=jnp.float32,
    )
    z2 = jnp.sum(zf * zf, axis=1, keepdims=True)

    lane = jax.lax.broadcasted_iota(jnp.int32, (1, _CH), 1).astype(jnp.float32)
    run_min = None
    run_idx = None
    for c in range(_NUM_CODES // _CH):
        lo, hi = c * _CH, (c + 1) * _CH
        sc = c2[:, lo:hi] + q2[:, lo:hi]
        idxc = lane + jnp.float32(c * _CH)
        if c == 0:
            run_min = sc
            run_idx = jnp.broadcast_to(idxc, sc.shape)
        else:
            lt = sc < run_min
            run_idx = jnp.where(lt, idxc, run_idx)
            run_min = jnp.minimum(run_min, sc)

    ms = jnp.min(run_min, axis=1, keepdims=True)
    cand = jnp.where(run_min == ms, run_idx, jnp.float32(1e9))
    arg = jnp.min(cand, axis=1, keepdims=True).astype(jnp.int32)
    m = z2 + ms
    ids = jnp.where(m <= _DIST_THRESHOLD, arg, jnp.int32(_NO_CODE_ID))
    ids_ref[...] = ids
    mind_ref[...] = m


def kernel(z, codes):
    Bv, Tv, D = z.shape
    n_tok = Bv * Tv
    zf = z.reshape(n_tok, D)
    codes_t = codes.T
    grid = (n_tok // _MT,)
    ids, mind = pl.pallas_call(
        _vq_kernel,
        grid=grid,
        in_specs=[
            pl.BlockSpec((_MT, D), lambda i: (i, 0)),
            pl.BlockSpec((_NUM_CODES, D), lambda i: (0, 0)),
            pl.BlockSpec((D, _NUM_CODES), lambda i: (0, 0)),
        ],
        out_specs=[
            pl.BlockSpec((_MT, 1), lambda i: (i, 0)),
            pl.BlockSpec((_MT, 1), lambda i: (i, 0)),
        ],
        out_shape=[
            jax.ShapeDtypeStruct((n_tok, 1), jnp.int32),
            jax.ShapeDtypeStruct((n_tok, 1), jnp.float32),
        ],
        compiler_params=pltpu.CompilerParams(
            dimension_semantics=("parallel",)),
    )(zf, codes, codes_t)
    return ids.reshape(Bv, Tv), mind.reshape(Bv, Tv)


# speedup vs baseline: 1.0445x; 1.0445x over previous
import jax
import jax.numpy as jnp
from jax.experimental import pallas as pl
from jax.experimental.pallas import tpu as pltpu

_NUM_CODES = 8192
_NO_CODE_ID = 8192
_DIST_THRESHOLD = 128.0
_CODE_DIM = 64
_MT = 1024
_CH = 128
_RB = 32


def _vq_kernel(z_ref, codes_ref, codes_t_ref, ids_ref, mind_ref, c2_ref):
    @pl.when(pl.program_id(0) == 0)
    def _():
        ct = codes_t_ref[...]
        c2_ref[...] = jnp.sum(ct * ct, axis=0, keepdims=True)

    c2 = c2_ref[...]
    zf = z_ref[...]
    zn = zf * jnp.float32(-2.0)
    q2 = jax.lax.dot_general(
        zn, codes_ref[...], (((1,), (1,)), ((), ())),
        preferred_element_type=jnp.float32,
    )
    z2 = jnp.sum(zf * zf, axis=1, keepdims=True)

    lane = jax.lax.broadcasted_iota(jnp.int32, (1, _CH), 1).astype(jnp.float32)
    run_min = None
    run_idx = None
    for c in range(_NUM_CODES // _CH):
        lo, hi = c * _CH, (c + 1) * _CH
        sc = c2[:, lo:hi] + q2[:, lo:hi]
        idxc = lane + jnp.float32(c * _CH)
        if c == 0:
            run_min = sc
            run_idx = jnp.broadcast_to(idxc, sc.shape)
        else:
            lt = sc < run_min
            run_idx = jnp.where(lt, idxc, run_idx)
            run_min = jnp.minimum(run_min, sc)

    ms = jnp.min(run_min, axis=1, keepdims=True)
    cand = jnp.where(run_min == ms, run_idx, jnp.float32(1e9))
    arg = jnp.min(cand, axis=1, keepdims=True).astype(jnp.int32)
    m = z2 + ms
    ids = jnp.where(m <= _DIST_THRESHOLD, arg, jnp.int32(_NO_CODE_ID))
    ids_ref[...] = ids
    mind_ref[...] = m


def kernel(z, codes):
    Bv, Tv, D = z.shape
    n_tok = Bv * Tv
    zf = z.reshape(n_tok, D)
    codes_t = codes.T
    grid = (n_tok // _MT,)
    ids, mind = pl.pallas_call(
        _vq_kernel,
        grid=grid,
        in_specs=[
            pl.BlockSpec((_MT, D), lambda i: (i, 0)),
            pl.BlockSpec((_NUM_CODES, D), lambda i: (0, 0)),
            pl.BlockSpec((D, _NUM_CODES), lambda i: (0, 0)),
        ],
        out_specs=[
            pl.BlockSpec((_MT, 1), lambda i: (i, 0)),
            pl.BlockSpec((_MT, 1), lambda i: (i, 0)),
        ],
        out_shape=[
            jax.ShapeDtypeStruct((n_tok, 1), jnp.int32),
            jax.ShapeDtypeStruct((n_tok, 1), jnp.float32),
        ],
        scratch_shapes=[pltpu.VMEM((1, _NUM_CODES), jnp.float32)],
    )(zf, codes, codes_t)
    return ids.reshape(Bv, Tv), mind.reshape(Bv, Tv)


# speedup vs baseline: 1.0485x; 1.0039x over previous
import jax
import jax.numpy as jnp
from jax.experimental import pallas as pl
from jax.experimental.pallas import tpu as pltpu

_NUM_CODES = 8192
_NO_CODE_ID = 8192
_DIST_THRESHOLD = 128.0
_CODE_DIM = 64
_MT = 1024
_CH = 128


def _vq_kernel(z_ref, codes_ref, codes_t_ref, ids_ref, mind_ref, c2_ref):
    @pl.when(pl.program_id(0) == 0)
    def _():
        ct = codes_t_ref[...]
        c2_ref[...] = jnp.sum(ct * ct, axis=0, keepdims=True)

    c2 = c2_ref[...]
    zf = z_ref[...]
    zn = zf * jnp.float32(-2.0)
    q2 = jax.lax.dot_general(
        zn, codes_ref[...], (((1,), (1,)), ((), ())),
        preferred_element_type=jnp.float32,
    )
    z2 = jnp.sum(zf * zf, axis=1, keepdims=True)

    lane = jax.lax.broadcasted_iota(jnp.int32, (1, _CH), 1).astype(jnp.float32)
    run_min = None
    run_idx = None
    for c in range(_NUM_CODES // _CH):
        lo, hi = c * _CH, (c + 1) * _CH
        sc = c2[:, lo:hi] + q2[:, lo:hi]
        idxc = lane + jnp.float32(c * _CH)
        if c == 0:
            run_min = sc
            run_idx = jnp.broadcast_to(idxc, sc.shape)
        else:
            lt = sc < run_min
            run_idx = jnp.where(lt, idxc, run_idx)
            run_min = jnp.minimum(run_min, sc)

    ms = jnp.min(run_min, axis=1, keepdims=True)
    cand = jnp.where(run_min == ms, run_idx, jnp.float32(1e9))
    arg = jnp.min(cand, axis=1, keepdims=True).astype(jnp.int32)
    m = z2 + ms
    ids = jnp.where(m <= _DIST_THRESHOLD, arg, jnp.int32(_NO_CODE_ID))
    ids_ref[...] = ids
    mind_ref[...] = m


def kernel(z, codes):
    Bv, Tv, D = z.shape
    n_tok = Bv * Tv
    zf = z.reshape(n_tok, D)
    codes_t = codes.T
    grid = (n_tok // _MT,)
    ids, mind = pl.pallas_call(
        _vq_kernel,
        grid=grid,
        in_specs=[
            pl.BlockSpec((_MT, D), lambda i: (i, 0)),
            pl.BlockSpec((_NUM_CODES, D), lambda i: (0, 0)),
            pl.BlockSpec((D, _NUM_CODES), lambda i: (0, 0)),
        ],
        out_specs=[
            pl.BlockSpec((_MT, 1), lambda i: (i, 0)),
            pl.BlockSpec((_MT, 1), lambda i: (i, 0)),
        ],
        out_shape=[
            jax.ShapeDtypeStruct((n_tok, 1), jnp.int32),
            jax.ShapeDtypeStruct((n_tok, 1), jnp.float32),
        ],
        scratch_shapes=[pltpu.VMEM((1, _NUM_CODES), jnp.float32)],
    )(zf, codes, codes_t)
    return ids.reshape(Bv, Tv), mind.reshape(Bv, Tv)
